# Initial kernel scaffold; baseline (speedup 1.0000x reference)
import jax
import jax.numpy as jnp
from jax.experimental import pallas as pl


def kernel(tokens, weight):
    raise NotImplementedError("write your pallas kernel here")


# speedup vs baseline: 1.0917x; 1.0917x over previous
import functools

import jax
import jax.numpy as jnp
from jax import lax
from jax.experimental import pallas as pl
from jax.experimental.pallas import tpu as pltpu
from jax.experimental.pallas import tpu_sc as plsc

PAD = 1
B, T, D = 4096, 200, 64
TP = 224
HALF = TP // 2
NW = 32
RPW = B // NW


def _make_sc_kernel():
    mesh = plsc.VectorSubcoreMesh(core_axis_name="c", subcore_axis_name="s")

    @functools.partial(
        pl.kernel,
        mesh=mesh,
        out_type=jax.ShapeDtypeStruct((B * T, D), jnp.float32),
        compiler_params=pltpu.CompilerParams(
            needs_layout_passes=False, use_tc_tiling_on_sc=False),
        scratch_types=[
            pltpu.VMEM((TP,), jnp.int32),
            pltpu.VMEM((2, HALF), jnp.int32),
            pltpu.VMEM((TP, D), jnp.float32),
            pltpu.SemaphoreType.DMA,
        ],
    )
    def k(tok_hbm, w_hbm, out_hbm, tok_v, idx_v, out_v, sem):
        wid = lax.axis_index("s") * 2 + lax.axis_index("c")
        base = wid * RPW

        def body(r, carry_none):
            row = base + r
            pltpu.sync_copy(tok_hbm.at[pl.ds(row * TP, TP)], tok_v)
            carry = jnp.int32(0)
            for c in range(TP // 16):
                t = tok_v[pl.ds(c * 16, 16)]
                m = jnp.where(t != PAD, jnp.int32(1), jnp.int32(0))
                cs = plsc.cumsum(m) + carry
                idx_v[c // 7, pl.ds((c % 7) * 16, 16)] = cs * m + PAD
                carry = carry + jnp.sum(m)
            pltpu.async_copy(w_hbm.at[idx_v.at[0]],
                             out_v.at[pl.ds(0, HALF)], sem).wait()
            pltpu.async_copy(w_hbm.at[idx_v.at[1]],
                             out_v.at[pl.ds(HALF, HALF)], sem).wait()
            pltpu.sync_copy(out_v.at[pl.ds(0, T)],
                            out_hbm.at[pl.ds(row * T, T)])
            return carry_none

        lax.fori_loop(0, RPW, body, 0)

    return k


def kernel(tokens, weight):
    tokens_p = jnp.pad(tokens.astype(jnp.int32), ((0, 0), (0, TP - T)),
                       constant_values=PAD)
    out = _make_sc_kernel()(tokens_p.reshape(-1), weight)
    return out.reshape(B, T, D)


# speedup vs baseline: 2.7645x; 2.5322x over previous
import functools

import jax
import jax.numpy as jnp
from jax import lax
from jax.experimental import pallas as pl
from jax.experimental.pallas import tpu as pltpu
from jax.experimental.pallas import tpu_sc as plsc

PAD = 1
B, T, D = 4096, 200, 64
TP = 224
HALF = TP // 2
REM = T - HALF
NW = 32
RPW = B // NW
NB = 4


def _make_sc_kernel():
    mesh = plsc.VectorSubcoreMesh(core_axis_name="c", subcore_axis_name="s")

    @functools.partial(
        pl.kernel,
        mesh=mesh,
        out_type=jax.ShapeDtypeStruct((B * T, D), jnp.float32),
        compiler_params=pltpu.CompilerParams(
            needs_layout_passes=False, use_tc_tiling_on_sc=False),
        scratch_types=[
            pltpu.VMEM((RPW * TP,), jnp.int32),
            pltpu.VMEM((2 * RPW, HALF), jnp.int32),
            pltpu.VMEM((NB, T, D), jnp.float32),
            pltpu.SemaphoreType.DMA,
            pltpu.SemaphoreType.DMA,
            pltpu.SemaphoreType.DMA,
            pltpu.SemaphoreType.DMA,
        ],
    )
    def k(tok_hbm, w_hbm, out_hbm, tok_v, idx_v, buf_v, s0, s1, s2, s3):
        sems = (s0, s1, s2, s3)
        wid = lax.axis_index("s") * 2 + lax.axis_index("c")
        base = wid * RPW

        pltpu.sync_copy(tok_hbm.at[pl.ds(base * TP, RPW * TP)], tok_v)

        def index_body(rl, carry_none):
            carry = jnp.int32(0)
            for c in range(TP // 16):
                t = tok_v[pl.ds(rl * TP + c * 16, 16)]
                m = jnp.where(t != PAD, jnp.int32(1), jnp.int32(0))
                cs = plsc.cumsum(m) + carry
                idx_v[2 * rl + c // 7, pl.ds((c % 7) * 16, 16)] = cs * m + PAD
                carry = carry + jnp.sum(m)
            return carry_none

        lax.fori_loop(0, RPW, index_body, 0)

        def fire(rl, b):
            pltpu.async_copy(w_hbm.at[idx_v.at[2 * rl]],
                             buf_v.at[b, pl.ds(0, HALF)], sems[b])
            pltpu.async_copy(w_hbm.at[idx_v.at[2 * rl + 1, pl.ds(0, REM)]],
                             buf_v.at[b, pl.ds(HALF, REM)], sems[b])

        def drain(rl, b):
            pltpu.make_async_copy(w_hbm.at[idx_v.at[2 * rl]],
                                  buf_v.at[b, pl.ds(0, HALF)], sems[b]).wait()
            pltpu.make_async_copy(w_hbm.at[idx_v.at[2 * rl + 1, pl.ds(0, REM)]],
                                  buf_v.at[b, pl.ds(HALF, REM)], sems[b]).wait()

        for b in range(NB):
            fire(b, b)

        def ring_body(g, carry_none):
            for b in range(NB):
                rl = g * NB + b
                drain(rl, b)
                pltpu.sync_copy(buf_v.at[b],
                                out_hbm.at[pl.ds((base + rl) * T, T)])
                fire(rl + NB, b)
            return carry_none

        lax.fori_loop(0, RPW // NB - 1, ring_body, 0)

        for b in range(NB):
            rl = RPW - NB + b
            drain(rl, b)
            pltpu.sync_copy(buf_v.at[b], out_hbm.at[pl.ds((base + rl) * T, T)])

    return k


def kernel(tokens, weight):
    tokens_p = jnp.pad(tokens.astype(jnp.int32), ((0, 0), (0, TP - T)),
                       constant_values=PAD)
    out = _make_sc_kernel()(tokens_p.reshape(-1), weight)
    return out.reshape(B, T, D)
